# Initial kernel scaffold; baseline (speedup 1.0000x reference)
import jax
import jax.numpy as jnp
from jax.experimental import pallas as pl


def kernel(z, codebook):
    raise NotImplementedError("write your pallas kernel here")


# speedup vs baseline: 1.0384x; 1.0384x over previous
import functools

import jax
import jax.numpy as jnp
from jax import lax
from jax.experimental import pallas as pl
from jax.experimental.pallas import tpu as pltpu
from jax.experimental.pallas import tpu_sc as plsc

K = 8192
CD = 256
BM = 512


BN = 1024
NJ = K // BN


def _dist_argmin_body(x_ref, cbt_ref, xsq_ref, csq_ref, idx_ref, loss_ref,
                      rmin_ref, ridx_ref):
    j = pl.program_id(0)
    i = pl.program_id(1)
    x = x_ref[...]
    ct = cbt_ref[...]
    xc = lax.dot_general(
        x.astype(jnp.bfloat16), ct.astype(jnp.bfloat16),
        dimension_numbers=(((1,), (0,)), ((), ())),
        preferred_element_type=jnp.float32,
    )
    scores = (xsq_ref[...] - 2.0 * xc) + csq_ref[...]
    cur_min = jnp.min(scores, axis=1, keepdims=True)
    n_iota = lax.broadcasted_iota(jnp.int32, scores.shape, 1) + j * BN
    cur_idx = jnp.min(jnp.where(scores == cur_min, n_iota, K),
                      axis=1, keepdims=True)
    row = pl.ds(i * BM, BM)

    @pl.when(j == 0)
    def _():
        rmin_ref[row, :] = cur_min
        ridx_ref[row, :] = cur_idx

    @pl.when(j > 0)
    def _():
        old_min = rmin_ref[row, :]
        old_idx = ridx_ref[row, :]
        better = cur_min < old_min
        rmin_ref[row, :] = jnp.where(better, cur_min, old_min)
        ridx_ref[row, :] = jnp.where(better, cur_idx, old_idx)

    @pl.when((j == NJ - 1) & (i == 0))
    def _():
        loss_ref[...] = jnp.zeros((1, 1), jnp.float32)

    @pl.when(j == NJ - 1)
    def _():
        idx_ref[0, 0, :] = ridx_ref[row, :][:, 0]
        loss_ref[...] += jnp.sum(rmin_ref[row, :], axis=0, keepdims=True)


def _nearest_codes(x, codebook):
    m = x.shape[0]
    n_tiles = m // BM
    x_sq = jnp.sum(x * x, axis=-1, keepdims=True)
    c_sq = jnp.sum(codebook * codebook, axis=-1)[None, :]
    idx3, loss = pl.pallas_call(
        _dist_argmin_body,
        grid=(NJ, n_tiles),
        in_specs=[
            pl.BlockSpec((BM, CD), lambda j, i: (i, 0)),
            pl.BlockSpec((CD, BN), lambda j, i: (0, j)),
            pl.BlockSpec((BM, 1), lambda j, i: (i, 0)),
            pl.BlockSpec((1, BN), lambda j, i: (0, j)),
        ],
        out_specs=[
            pl.BlockSpec((1, 1, BM), lambda j, i: (i, 0, 0)),
            pl.BlockSpec((1, 1), lambda j, i: (0, 0)),
        ],
        out_shape=[
            jax.ShapeDtypeStruct((n_tiles, 1, BM), jnp.int32),
            jax.ShapeDtypeStruct((1, 1), jnp.float32),
        ],
        scratch_shapes=[
            pltpu.VMEM((m, 1), jnp.float32),
            pltpu.VMEM((m, 1), jnp.int32),
        ],
    )(x, codebook.T, x_sq, c_sq)
    return idx3.reshape(m), loss[0, 0]


def _make_sc_gather(n_rows):
    info = plsc.get_sparse_core_info()
    nw = info.num_cores * info.num_subcores
    b_per_w = n_rows // nw
    chunk = 144
    n_chunks = b_per_w // chunk
    mesh = plsc.VectorSubcoreMesh(core_axis_name="c", subcore_axis_name="s")

    @functools.partial(
        pl.kernel,
        mesh=mesh,
        out_type=jax.ShapeDtypeStruct((n_rows, CD), jnp.float32),
        scratch_types=[
            pltpu.VMEM((b_per_w,), jnp.int32),
            pltpu.VMEM((chunk, CD), jnp.float32),
            pltpu.SemaphoreType.DMA,
        ],
    )
    def gather_rows(table_hbm, idx_hbm, out_hbm, idx_v, rows_v, sem):
        wid = lax.axis_index("s") * info.num_cores + lax.axis_index("c")
        base = wid * b_per_w
        pltpu.sync_copy(idx_hbm.at[pl.ds(base, b_per_w)], idx_v)
        for cix in range(n_chunks):
            pltpu.async_copy(
                table_hbm.at[idx_v.at[pl.ds(cix * chunk, chunk)]],
                rows_v, sem).wait()
            pltpu.sync_copy(rows_v, out_hbm.at[pl.ds(base + cix * chunk, chunk)])

    return gather_rows


def kernel(z, codebook):
    cd = codebook.shape[1]
    shp = z.shape
    z_grouped = z.reshape(shp[:-1] + (-1, cd))
    orig_shape = z_grouped.shape
    x = z_grouped.reshape((-1, cd))

    indices_flat, loss_sum = _nearest_codes(x, codebook)
    quantize = _make_sc_gather(x.shape[0])(codebook, indices_flat)

    commit_loss = loss_sum / jnp.float32(x.size)
    indices = indices_flat.reshape(orig_shape[:-1])
    codes = quantize.reshape(orig_shape[:-2] + (orig_shape[-2] * cd,))
    return codes, indices, z_grouped, commit_loss


# speedup vs baseline: 1.3592x; 1.3089x over previous
import functools

import jax
import jax.numpy as jnp
from jax import lax
from jax.experimental import pallas as pl
from jax.experimental.pallas import tpu as pltpu
from jax.experimental.pallas import tpu_sc as plsc

K = 8192
CD = 256
BM = 512


BN = 4096
NJ = K // BN


def _dist_argmin_body(xm2_ref, cbt_ref, xsq_ref, csq_ref, idx_ref, loss_ref,
                      rmin_ref, ridx_ref):
    j = pl.program_id(0)
    i = pl.program_id(1)
    xc2 = lax.dot_general(
        xm2_ref[...], cbt_ref[...],
        dimension_numbers=(((1,), (0,)), ((), ())),
        preferred_element_type=jnp.float32,
    )
    scores = (xsq_ref[...] + xc2) + csq_ref[...]
    cur_min = jnp.min(scores, axis=1, keepdims=True)
    n_iota = (lax.broadcasted_iota(jnp.int32, (1, BN), 1).astype(jnp.float32)
              + jnp.float32(j * BN))
    cur_idx = jnp.min(jnp.where(scores == cur_min, n_iota, jnp.float32(K)),
                      axis=1, keepdims=True)
    row = pl.ds(i * BM, BM)

    @pl.when(j == 0)
    def _():
        rmin_ref[row, :] = cur_min
        ridx_ref[row, :] = cur_idx

    @pl.when(j > 0)
    def _():
        old_min = rmin_ref[row, :]
        old_idx = ridx_ref[row, :]
        better = cur_min < old_min
        rmin_ref[row, :] = jnp.where(better, cur_min, old_min)
        ridx_ref[row, :] = jnp.where(better, cur_idx, old_idx)

    @pl.when((j == NJ - 1) & (i == 0))
    def _():
        loss_ref[...] = jnp.zeros((1, 1), jnp.float32)

    @pl.when(j == NJ - 1)
    def _():
        idx_ref[0, 0, :] = ridx_ref[row, :][:, 0].astype(jnp.int32)
        loss_ref[...] += jnp.sum(rmin_ref[row, :], axis=0, keepdims=True)


def _nearest_codes(x, codebook):
    m = x.shape[0]
    n_tiles = m // BM
    x_sq = jnp.sum(x * x, axis=-1, keepdims=True)
    c_sq = jnp.sum(codebook * codebook, axis=-1)[None, :]
    xm2 = (-2.0 * x).astype(jnp.bfloat16)
    cbt = codebook.T.astype(jnp.bfloat16)
    idx3, loss = pl.pallas_call(
        _dist_argmin_body,
        grid=(NJ, n_tiles),
        in_specs=[
            pl.BlockSpec((BM, CD), lambda j, i: (i, 0)),
            pl.BlockSpec((CD, BN), lambda j, i: (0, j)),
            pl.BlockSpec((BM, 1), lambda j, i: (i, 0)),
            pl.BlockSpec((1, BN), lambda j, i: (0, j)),
        ],
        out_specs=[
            pl.BlockSpec((1, 1, BM), lambda j, i: (i, 0, 0)),
            pl.BlockSpec((1, 1), lambda j, i: (0, 0)),
        ],
        out_shape=[
            jax.ShapeDtypeStruct((n_tiles, 1, BM), jnp.int32),
            jax.ShapeDtypeStruct((1, 1), jnp.float32),
        ],
        scratch_shapes=[
            pltpu.VMEM((m, 1), jnp.float32),
            pltpu.VMEM((m, 1), jnp.float32),
        ],
    )(xm2, cbt, x_sq, c_sq)
    return idx3.reshape(m), loss[0, 0]


def _make_sc_gather(n_rows):
    info = plsc.get_sparse_core_info()
    nw = info.num_cores * info.num_subcores
    b_per_w = n_rows // nw
    chunk = 144
    n_chunks = b_per_w // chunk
    mesh = plsc.VectorSubcoreMesh(core_axis_name="c", subcore_axis_name="s")

    @functools.partial(
        pl.kernel,
        mesh=mesh,
        out_type=jax.ShapeDtypeStruct((n_rows, CD), jnp.float32),
        scratch_types=[
            pltpu.VMEM((b_per_w,), jnp.int32),
            pltpu.VMEM((chunk, CD), jnp.float32),
            pltpu.SemaphoreType.DMA,
        ],
    )
    def gather_rows(table_hbm, idx_hbm, out_hbm, idx_v, rows_v, sem):
        wid = lax.axis_index("s") * info.num_cores + lax.axis_index("c")
        base = wid * b_per_w
        pltpu.sync_copy(idx_hbm.at[pl.ds(base, b_per_w)], idx_v)
        for cix in range(n_chunks):
            pltpu.async_copy(
                table_hbm.at[idx_v.at[pl.ds(cix * chunk, chunk)]],
                rows_v, sem).wait()
            pltpu.sync_copy(rows_v, out_hbm.at[pl.ds(base + cix * chunk, chunk)])

    return gather_rows


def kernel(z, codebook):
    cd = codebook.shape[1]
    shp = z.shape
    z_grouped = z.reshape(shp[:-1] + (-1, cd))
    orig_shape = z_grouped.shape
    x = z_grouped.reshape((-1, cd))

    indices_flat, loss_sum = _nearest_codes(x, codebook)
    quantize = _make_sc_gather(x.shape[0])(codebook, indices_flat)

    commit_loss = loss_sum / jnp.float32(x.size)
    indices = indices_flat.reshape(orig_shape[:-1])
    codes = quantize.reshape(orig_shape[:-2] + (orig_shape[-2] * cd,))
    return codes, indices, z_grouped, commit_loss


# speedup vs baseline: 1.4108x; 1.0380x over previous
import functools

import jax
import jax.numpy as jnp
from jax import lax
from jax.experimental import pallas as pl
from jax.experimental.pallas import tpu as pltpu
from jax.experimental.pallas import tpu_sc as plsc

K = 8192
CD = 256
BM = 512


BN = 8192
NJ = K // BN


def _dist_argmin_body(xm2_ref, cbt_ref, xsq_ref, csq_ref, idx_ref, loss_ref,
                      rmin_ref, ridx_ref):
    j = pl.program_id(0)
    i = pl.program_id(1)
    xc2 = lax.dot_general(
        xm2_ref[...], cbt_ref[...],
        dimension_numbers=(((1,), (0,)), ((), ())),
        preferred_element_type=jnp.float32,
    )
    scores = (xsq_ref[...] + xc2) + csq_ref[...]
    cur_min = jnp.min(scores, axis=1, keepdims=True)
    n_iota = (lax.broadcasted_iota(jnp.int32, (1, BN), 1).astype(jnp.float32)
              + jnp.float32(j * BN))
    cur_idx = jnp.min(jnp.where(scores == cur_min, n_iota, jnp.float32(K)),
                      axis=1, keepdims=True)
    row = pl.ds(i * BM, BM)

    @pl.when(j == 0)
    def _():
        rmin_ref[row, :] = cur_min
        ridx_ref[row, :] = cur_idx

    @pl.when(j > 0)
    def _():
        old_min = rmin_ref[row, :]
        old_idx = ridx_ref[row, :]
        better = cur_min < old_min
        rmin_ref[row, :] = jnp.where(better, cur_min, old_min)
        ridx_ref[row, :] = jnp.where(better, cur_idx, old_idx)

    @pl.when((j == NJ - 1) & (i == 0))
    def _():
        loss_ref[...] = jnp.zeros((1, 1), jnp.float32)

    @pl.when(j == NJ - 1)
    def _():
        idx_ref[0, 0, :] = ridx_ref[row, :][:, 0].astype(jnp.int32)
        loss_ref[...] += jnp.sum(rmin_ref[row, :], axis=0, keepdims=True)


def _nearest_codes(x, codebook):
    m = x.shape[0]
    n_tiles = m // BM
    x_sq = jnp.sum(x * x, axis=-1, keepdims=True)
    c_sq = jnp.sum(codebook * codebook, axis=-1)[None, :]
    xm2 = (-2.0 * x).astype(jnp.bfloat16)
    cbt = codebook.T.astype(jnp.bfloat16)
    idx3, loss = pl.pallas_call(
        _dist_argmin_body,
        grid=(NJ, n_tiles),
        in_specs=[
            pl.BlockSpec((BM, CD), lambda j, i: (i, 0)),
            pl.BlockSpec((CD, BN), lambda j, i: (0, j)),
            pl.BlockSpec((BM, 1), lambda j, i: (i, 0)),
            pl.BlockSpec((1, BN), lambda j, i: (0, j)),
        ],
        out_specs=[
            pl.BlockSpec((1, 1, BM), lambda j, i: (i, 0, 0)),
            pl.BlockSpec((1, 1), lambda j, i: (0, 0)),
        ],
        out_shape=[
            jax.ShapeDtypeStruct((n_tiles, 1, BM), jnp.int32),
            jax.ShapeDtypeStruct((1, 1), jnp.float32),
        ],
        scratch_shapes=[
            pltpu.VMEM((m, 1), jnp.float32),
            pltpu.VMEM((m, 1), jnp.float32),
        ],
    )(xm2, cbt, x_sq, c_sq)
    return idx3.reshape(m), loss[0, 0]


def _make_sc_gather(n_rows):
    info = plsc.get_sparse_core_info()
    nw = info.num_cores * info.num_subcores
    b_per_w = n_rows // nw
    chunk = 144
    n_chunks = b_per_w // chunk
    mesh = plsc.VectorSubcoreMesh(core_axis_name="c", subcore_axis_name="s")

    @functools.partial(
        pl.kernel,
        mesh=mesh,
        out_type=jax.ShapeDtypeStruct((n_rows, CD), jnp.float32),
        scratch_types=[
            pltpu.VMEM((b_per_w,), jnp.int32),
            pltpu.VMEM((chunk, CD), jnp.float32),
            pltpu.SemaphoreType.DMA,
        ],
    )
    def gather_rows(table_hbm, idx_hbm, out_hbm, idx_v, rows_v, sem):
        wid = lax.axis_index("s") * info.num_cores + lax.axis_index("c")
        base = wid * b_per_w
        pltpu.sync_copy(idx_hbm.at[pl.ds(base, b_per_w)], idx_v)
        for cix in range(n_chunks):
            pltpu.async_copy(
                table_hbm.at[idx_v.at[pl.ds(cix * chunk, chunk)]],
                rows_v, sem).wait()
            pltpu.sync_copy(rows_v, out_hbm.at[pl.ds(base + cix * chunk, chunk)])

    return gather_rows


def kernel(z, codebook):
    cd = codebook.shape[1]
    shp = z.shape
    z_grouped = z.reshape(shp[:-1] + (-1, cd))
    orig_shape = z_grouped.shape
    x = z_grouped.reshape((-1, cd))

    indices_flat, loss_sum = _nearest_codes(x, codebook)
    quantize = _make_sc_gather(x.shape[0])(codebook, indices_flat)

    commit_loss = loss_sum / jnp.float32(x.size)
    indices = indices_flat.reshape(orig_shape[:-1])
    codes = quantize.reshape(orig_shape[:-2] + (orig_shape[-2] * cd,))
    return codes, indices, z_grouped, commit_loss


# speedup vs baseline: 1.4192x; 1.0060x over previous
import functools

import jax
import jax.numpy as jnp
from jax import lax
from jax.experimental import pallas as pl
from jax.experimental.pallas import tpu as pltpu
from jax.experimental.pallas import tpu_sc as plsc

K = 8192
CD = 256
BM = 512


def _dist_argmin_body(xm2_ref, cb_ref, xsq_ref, csq_ref, idx_ref, loss_ref):
    i = pl.program_id(0)
    xc2 = lax.dot_general(
        xm2_ref[...], cb_ref[...],
        dimension_numbers=(((1,), (1,)), ((), ())),
        preferred_element_type=jnp.float32,
    )
    scores = (xsq_ref[...] + xc2) + csq_ref[...]
    cur_min = jnp.min(scores, axis=1, keepdims=True)
    n_iota = lax.broadcasted_iota(jnp.int32, (1, K), 1).astype(jnp.float32)
    cur_idx = jnp.min(jnp.where(scores == cur_min, n_iota, jnp.float32(K)),
                      axis=1, keepdims=True)
    idx_ref[0, 0, :] = cur_idx[:, 0].astype(jnp.int32)

    @pl.when(i == 0)
    def _():
        loss_ref[...] = jnp.zeros((1, 1), jnp.float32)

    loss_ref[...] += jnp.sum(cur_min, axis=0, keepdims=True)


def _nearest_codes(x, codebook):
    m = x.shape[0]
    n_tiles = m // BM
    x_sq = jnp.sum(x * x, axis=-1, keepdims=True)
    c_sq = jnp.sum(codebook * codebook, axis=-1)[None, :]
    xm2 = (-2.0 * x).astype(jnp.bfloat16)
    cb = codebook.astype(jnp.bfloat16)
    idx3, loss = pl.pallas_call(
        _dist_argmin_body,
        grid=(n_tiles,),
        in_specs=[
            pl.BlockSpec((BM, CD), lambda i: (i, 0)),
            pl.BlockSpec((K, CD), lambda i: (0, 0)),
            pl.BlockSpec((BM, 1), lambda i: (i, 0)),
            pl.BlockSpec((1, K), lambda i: (0, 0)),
        ],
        out_specs=[
            pl.BlockSpec((1, 1, BM), lambda i: (i, 0, 0)),
            pl.BlockSpec((1, 1), lambda i: (0, 0)),
        ],
        out_shape=[
            jax.ShapeDtypeStruct((n_tiles, 1, BM), jnp.int32),
            jax.ShapeDtypeStruct((1, 1), jnp.float32),
        ],
    )(xm2, cb, x_sq, c_sq)
    return idx3.reshape(m), loss[0, 0]


def _make_sc_gather(n_rows):
    info = plsc.get_sparse_core_info()
    nw = info.num_cores * info.num_subcores
    b_per_w = n_rows // nw
    chunk = 144
    n_chunks = b_per_w // chunk
    mesh = plsc.VectorSubcoreMesh(core_axis_name="c", subcore_axis_name="s")

    @functools.partial(
        pl.kernel,
        mesh=mesh,
        out_type=jax.ShapeDtypeStruct((n_rows, CD), jnp.float32),
        scratch_types=[
            pltpu.VMEM((b_per_w,), jnp.int32),
            pltpu.VMEM((chunk, CD), jnp.float32),
            pltpu.SemaphoreType.DMA,
        ],
    )
    def gather_rows(table_hbm, idx_hbm, out_hbm, idx_v, rows_v, sem):
        wid = lax.axis_index("s") * info.num_cores + lax.axis_index("c")
        base = wid * b_per_w
        pltpu.sync_copy(idx_hbm.at[pl.ds(base, b_per_w)], idx_v)
        for cix in range(n_chunks):
            pltpu.async_copy(
                table_hbm.at[idx_v.at[pl.ds(cix * chunk, chunk)]],
                rows_v, sem).wait()
            pltpu.sync_copy(rows_v, out_hbm.at[pl.ds(base + cix * chunk, chunk)])

    return gather_rows


def kernel(z, codebook):
    cd = codebook.shape[1]
    shp = z.shape
    z_grouped = z.reshape(shp[:-1] + (-1, cd))
    orig_shape = z_grouped.shape
    x = z_grouped.reshape((-1, cd))

    indices_flat, loss_sum = _nearest_codes(x, codebook)
    quantize = _make_sc_gather(x.shape[0])(codebook, indices_flat)

    commit_loss = loss_sum / jnp.float32(x.size)
    indices = indices_flat.reshape(orig_shape[:-1])
    codes = quantize.reshape(orig_shape[:-2] + (orig_shape[-2] * cd,))
    return codes, indices, z_grouped, commit_loss


# speedup vs baseline: 1.5505x; 1.0925x over previous
import functools

import jax
import jax.numpy as jnp
from jax import lax
from jax.experimental import pallas as pl
from jax.experimental.pallas import tpu as pltpu
from jax.experimental.pallas import tpu_sc as plsc

K = 8192
CD = 256
BM = 512


def _dist_argmin_body(x_ref, cb_ref, xsq_ref, csq_ref, idx_ref, loss_ref):
    i = pl.program_id(0)
    xm2 = (-2.0 * x_ref[...]).astype(jnp.bfloat16)
    xc2 = lax.dot_general(
        xm2, cb_ref[...],
        dimension_numbers=(((1,), (1,)), ((), ())),
        preferred_element_type=jnp.float32,
    )
    scores = (xsq_ref[...] + xc2) + csq_ref[...]
    cur_min = jnp.min(scores, axis=1, keepdims=True)
    n_iota = lax.broadcasted_iota(jnp.int32, (1, K), 1).astype(jnp.float32)
    cur_idx = jnp.min(jnp.where(scores == cur_min, n_iota, jnp.float32(K)),
                      axis=1, keepdims=True)
    idx_ref[0, 0, :] = cur_idx[:, 0].astype(jnp.int32)

    @pl.when(i == 0)
    def _():
        loss_ref[...] = jnp.zeros((1, 1), jnp.float32)

    loss_ref[...] += jnp.sum(cur_min, axis=0, keepdims=True)


def _nearest_codes(x, codebook):
    m = x.shape[0]
    n_tiles = m // BM
    x_sq = jnp.sum(x * x, axis=-1, keepdims=True)
    c_sq = jnp.sum(codebook * codebook, axis=-1)[None, :]
    cb = codebook.astype(jnp.bfloat16)
    idx3, loss = pl.pallas_call(
        _dist_argmin_body,
        grid=(n_tiles,),
        in_specs=[
            pl.BlockSpec((BM, CD), lambda i: (i, 0)),
            pl.BlockSpec((K, CD), lambda i: (0, 0)),
            pl.BlockSpec((BM, 1), lambda i: (i, 0)),
            pl.BlockSpec((1, K), lambda i: (0, 0)),
        ],
        out_specs=[
            pl.BlockSpec((1, 1, BM), lambda i: (i, 0, 0)),
            pl.BlockSpec((1, 1), lambda i: (0, 0)),
        ],
        out_shape=[
            jax.ShapeDtypeStruct((n_tiles, 1, BM), jnp.int32),
            jax.ShapeDtypeStruct((1, 1), jnp.float32),
        ],
    )(x, cb, x_sq, c_sq)
    return idx3.reshape(m), loss[0, 0]


def _make_sc_gather(n_rows):
    info = plsc.get_sparse_core_info()
    nw = info.num_cores * info.num_subcores
    b_per_w = n_rows // nw
    chunk = 144
    n_chunks = b_per_w // chunk
    mesh = plsc.VectorSubcoreMesh(core_axis_name="c", subcore_axis_name="s")

    @functools.partial(
        pl.kernel,
        mesh=mesh,
        out_type=jax.ShapeDtypeStruct((n_rows, CD), jnp.float32),
        scratch_types=[
            pltpu.VMEM((b_per_w,), jnp.int32),
            pltpu.VMEM((chunk, CD), jnp.float32),
            pltpu.VMEM((chunk, CD), jnp.float32),
            pltpu.SemaphoreType.DMA,
            pltpu.SemaphoreType.DMA,
        ],
    )
    def gather_rows(table_hbm, idx_hbm, out_hbm, idx_v, rows_a, rows_b, sem_a,
                    sem_b):
        wid = lax.axis_index("s") * info.num_cores + lax.axis_index("c")
        base = wid * b_per_w
        pltpu.sync_copy(idx_hbm.at[pl.ds(base, b_per_w)], idx_v)
        bufs = (rows_a, rows_b)
        sems = (sem_a, sem_b)
        copies = [None] * n_chunks
        copies[0] = pltpu.async_copy(
            table_hbm.at[idx_v.at[pl.ds(0, chunk)]], bufs[0], sems[0])
        for cix in range(n_chunks):
            nxt = cix + 1
            if nxt < n_chunks:
                copies[nxt] = pltpu.async_copy(
                    table_hbm.at[idx_v.at[pl.ds(nxt * chunk, chunk)]],
                    bufs[nxt % 2], sems[nxt % 2])
            copies[cix].wait()
            pltpu.sync_copy(bufs[cix % 2],
                            out_hbm.at[pl.ds(base + cix * chunk, chunk)])

    return gather_rows


def kernel(z, codebook):
    cd = codebook.shape[1]
    shp = z.shape
    z_grouped = z.reshape(shp[:-1] + (-1, cd))
    orig_shape = z_grouped.shape
    x = z_grouped.reshape((-1, cd))

    indices_flat, loss_sum = _nearest_codes(x, codebook)
    quantize = _make_sc_gather(x.shape[0])(codebook, indices_flat)

    commit_loss = loss_sum / jnp.float32(x.size)
    indices = indices_flat.reshape(orig_shape[:-1])
    codes = quantize.reshape(orig_shape[:-2] + (orig_shape[-2] * cd,))
    return codes, indices, z_grouped, commit_loss
